# Initial kernel scaffold; baseline (speedup 1.0000x reference)
import jax
import jax.numpy as jnp
from jax.experimental import pallas as pl


def kernel(x, pos, edge_index, batch, W1, b1, W2, b2, Wc0, al0, ar0, Wc1, al1, ar1, Wc2, al2, ar2, Wl1, bl1, Wl2, bl2):
    raise NotImplementedError("write your pallas kernel here")


# speedup vs baseline: 46.8819x; 46.8819x over previous
import functools

import jax
import jax.numpy as jnp
from jax import lax
from jax.experimental import pallas as pl
from jax.experimental.pallas import tpu as pltpu
from jax.experimental.pallas import tpu_sc as plsc

N = 10000
E = 160000
IN = 128
HID = 16
H = 16
C = 16
OUT = 128
NB = 64
NEG = 0.1

NC = 2
NS = 16
NW = NC * NS
K = 128
NCHUNK = E // K
NPAD = 10240
ZR = NPAD // NS

RB = 1000
NRB = N // RB

_f32 = jnp.float32


def _enc_body(xb, w1, b1, w2, b2, ob):
    t = jnp.maximum(
        jnp.dot(xb[...], w1[...], preferred_element_type=_f32) + b1[...], 0.0)
    ob[...] = jnp.dot(t, w2[...], preferred_element_type=_f32) + b2[...]


def _encoder(hcat, w1t, b1, w2t, b2):
    kin = hcat.shape[1]
    return pl.pallas_call(
        _enc_body,
        grid=(NRB,),
        in_specs=[
            pl.BlockSpec((RB, kin), lambda i: (i, 0)),
            pl.BlockSpec((kin, HID), lambda i: (0, 0)),
            pl.BlockSpec((1, HID), lambda i: (0, 0)),
            pl.BlockSpec((HID, HID), lambda i: (0, 0)),
            pl.BlockSpec((1, HID), lambda i: (0, 0)),
        ],
        out_specs=pl.BlockSpec((RB, HID), lambda i: (i, 0)),
        out_shape=jax.ShapeDtypeStruct((N, HID), _f32),
    )(hcat, w1t, b1, w2t, b2)


def _pre_first_body(hb, wc, al, ar, xl_o, al_o, ar_o):
    xl = jnp.dot(hb[...], wc[...], preferred_element_type=_f32)
    xl_o[...] = xl
    al_o[...] = jnp.dot(xl, al[...], preferred_element_type=_f32)
    ar_o[...] = jnp.dot(xl, ar[...], preferred_element_type=_f32)


def _pre_next_body(p0b, p1b, wc, al, ar, xl_o, al_o, ar_o):
    h = jnp.maximum(p0b[...] + p1b[...], 0.0)
    xl = jnp.dot(h, wc[...], preferred_element_type=_f32)
    xl_o[...] = xl
    al_o[...] = jnp.dot(xl, al[...], preferred_element_type=_f32)
    ar_o[...] = jnp.dot(xl, ar[...], preferred_element_type=_f32)


def _pre_specs(n_h_inputs):
    in_specs = [pl.BlockSpec((RB, HID), lambda i: (i, 0))] * n_h_inputs + [
        pl.BlockSpec((HID, H * C), lambda i: (0, 0)),
        pl.BlockSpec((H * C, H), lambda i: (0, 0)),
        pl.BlockSpec((H * C, H), lambda i: (0, 0)),
    ]
    out_specs = [
        pl.BlockSpec((RB, H * C), lambda i: (i, 0)),
        pl.BlockSpec((RB, H), lambda i: (i, 0)),
        pl.BlockSpec((RB, H), lambda i: (i, 0)),
    ]
    out_shape = [
        jax.ShapeDtypeStruct((N, H * C), _f32),
        jax.ShapeDtypeStruct((N, H), _f32),
        jax.ShapeDtypeStruct((N, H), _f32),
    ]
    return in_specs, out_specs, out_shape


def _pre_first(h, wct, almat, armat):
    ins, outs, oshape = _pre_specs(1)
    return pl.pallas_call(
        _pre_first_body, grid=(NRB,), in_specs=ins, out_specs=outs,
        out_shape=oshape)(h, wct, almat, armat)


def _pre_next(p0, p1, wct, almat, armat):
    ins, outs, oshape = _pre_specs(2)
    return pl.pallas_call(
        _pre_next_body, grid=(NRB,), in_specs=ins, out_specs=outs,
        out_shape=oshape)(p0, p1, wct, almat, armat)


def _dec_body(p0b, p1b, bb, w1, b1, w2, b2, ob):
    i = pl.program_id(0)
    h = jnp.maximum(p0b[...] + p1b[...], 0.0)
    t = jnp.maximum(
        jnp.dot(h, w1[...], preferred_element_type=_f32) + b1[...], 0.0)
    y = jnp.dot(t, w2[...], preferred_element_type=_f32) + b2[...]
    bidx = bb[0]
    oh = (lax.broadcasted_iota(jnp.int32, (NB, RB), 0) == bidx).astype(_f32)
    contrib = jnp.dot(oh, y, preferred_element_type=_f32)

    @pl.when(i == 0)
    def _():
        ob[...] = contrib

    @pl.when(i > 0)
    def _():
        ob[...] = ob[...] + contrib


def _decoder(p0, p1, batch3, w1t, b1, w2t, b2):
    return pl.pallas_call(
        _dec_body,
        grid=(NRB,),
        in_specs=[
            pl.BlockSpec((RB, HID), lambda i: (i, 0)),
            pl.BlockSpec((RB, HID), lambda i: (i, 0)),
            pl.BlockSpec((1, 1, RB), lambda i: (i, 0, 0)),
            pl.BlockSpec((HID, HID // 2), lambda i: (0, 0)),
            pl.BlockSpec((1, HID // 2), lambda i: (0, 0)),
            pl.BlockSpec((HID // 2, OUT), lambda i: (0, 0)),
            pl.BlockSpec((1, OUT), lambda i: (0, 0)),
        ],
        out_specs=pl.BlockSpec((NB, OUT), lambda i: (0, 0)),
        out_shape=jax.ShapeDtypeStruct((NB, OUT), _f32),
        compiler_params=pltpu.CompilerParams(
            dimension_semantics=("arbitrary",)),
    )(p0, p1, batch3, w1t, b1, w2t, b2)


_MESH = plsc.VectorSubcoreMesh(core_axis_name="c", subcore_axis_name="s")


@functools.partial(
    pl.kernel,
    out_type=(
        jax.ShapeDtypeStruct((E, H), _f32),
        jax.ShapeDtypeStruct((NC, NPAD, H), _f32),
    ),
    mesh=_MESH,
    compiler_params=pltpu.CompilerParams(use_tc_tiling_on_sc=False),
    scratch_types=[
        pltpu.VMEM((K,), jnp.int32),
        pltpu.VMEM((K,), jnp.int32),
        pltpu.VMEM((K, H), _f32),
        pltpu.VMEM((K, H), _f32),
        pltpu.VMEM_SHARED((NPAD, H), _f32),
        pltpu.VMEM_SHARED((NPAD, H), _f32),
        pltpu.VMEM_SHARED((NPAD, H), _f32),
    ],
)
def _sc_pass_a(al_hbm, ar_hbm, src_hbm, dst_hbm, zeros_hbm,
               ex_hbm, den_hbm,
               idx_s, idx_d, abuf, ebuf, al_sh, ar_sh, den_sh):
    cid = lax.axis_index("c")
    sid = lax.axis_index("s")
    wid = sid * NC + cid
    stripe = pl.ds(sid * ZR, ZR)

    pltpu.sync_copy(al_hbm.at[stripe], al_sh.at[stripe])
    pltpu.sync_copy(ar_hbm.at[stripe], ar_sh.at[stripe])
    pltpu.sync_copy(zeros_hbm.at[stripe], den_sh.at[stripe])
    plsc.subcore_barrier()

    nch = NCHUNK // NW + jnp.where(wid < (NCHUNK % NW), 1, 0)

    def chunk(j, carry):
        base = (wid + NW * j) * K
        pltpu.sync_copy(src_hbm.at[pl.ds(base, K)], idx_s)
        pltpu.sync_copy(dst_hbm.at[pl.ds(base, K)], idx_d)
        pltpu.sync_copy(al_sh.at[idx_s], abuf)
        pltpu.sync_copy(ar_sh.at[idx_d], ebuf)

        def edge(e, c):
            v = abuf[e] + ebuf[e]
            v = jnp.maximum(v, v * NEG)
            ebuf[e] = jnp.exp(v)
            return c

        lax.fori_loop(0, K, edge, 0)
        pltpu.sync_copy(ebuf, ex_hbm.at[pl.ds(base, K)])
        pltpu.sync_copy(ebuf, den_sh.at[idx_d], add=True)
        return carry

    lax.fori_loop(0, nch, chunk, 0)
    plsc.subcore_barrier()
    pltpu.sync_copy(den_sh.at[stripe], den_hbm.at[cid, stripe])


@functools.partial(
    pl.kernel,
    out_type=jax.ShapeDtypeStruct((NC, NPAD, C), _f32),
    mesh=_MESH,
    compiler_params=pltpu.CompilerParams(use_tc_tiling_on_sc=False),
    scratch_types=[
        pltpu.VMEM((K,), jnp.int32),
        pltpu.VMEM((K,), jnp.int32),
        pltpu.VMEM((K, H), _f32),
        pltpu.VMEM((K, H), _f32),
        pltpu.VMEM((K, H * C), _f32),
        pltpu.VMEM((K, C), _f32),
        pltpu.VMEM((ZR, H), _f32),
        pltpu.VMEM((ZR, H), _f32),
        pltpu.SemaphoreType.DMA,
        pltpu.VMEM_SHARED((NPAD, H), _f32),
        pltpu.VMEM_SHARED((NPAD, C), _f32),
    ],
)
def _sc_pass_b(xl_hbm, ex_hbm, den0_hbm, den1_hbm, src_hbm, dst_hbm,
               zeros_hbm, out_hbm,
               idx_s, idx_d, exbuf, dbuf, xlb, mb, v0, v1,
               sem1, den_sh, out_sh):
    cid = lax.axis_index("c")
    sid = lax.axis_index("s")
    wid = sid * NC + cid
    stripe = pl.ds(sid * ZR, ZR)

    pltpu.sync_copy(den0_hbm.at[stripe], v0)
    pltpu.sync_copy(den1_hbm.at[stripe], v1)

    def addrow(e, c):
        v0[e] = v0[e] + v1[e]
        return c

    lax.fori_loop(0, ZR, addrow, 0)
    pltpu.sync_copy(v0, den_sh.at[stripe])
    pltpu.sync_copy(zeros_hbm.at[stripe], out_sh.at[stripe])
    plsc.subcore_barrier()

    nch = NCHUNK // NW + jnp.where(wid < (NCHUNK % NW), 1, 0)

    def chunk(j, carry):
        base = (wid + NW * j) * K
        pltpu.sync_copy(src_hbm.at[pl.ds(base, K)], idx_s)
        pltpu.sync_copy(dst_hbm.at[pl.ds(base, K)], idx_d)
        g1 = pltpu.async_copy(xl_hbm.at[idx_s], xlb, sem1)
        pltpu.sync_copy(ex_hbm.at[pl.ds(base, K)], exbuf)
        pltpu.sync_copy(den_sh.at[idx_d], dbuf)
        g1.wait()

        def edge(e, c):
            w = exbuf[e] / (dbuf[e] + 1e-16)
            acc = w[0] * xlb[e, pl.ds(0, C)]
            for h in range(1, H):
                acc = acc + w[h] * xlb[e, pl.ds(h * C, C)]
            mb[e] = acc * (1.0 / H)
            return c

        lax.fori_loop(0, K, edge, 0)
        pltpu.sync_copy(mb, out_sh.at[idx_d], add=True)
        return carry

    lax.fori_loop(0, nch, chunk, 0)
    plsc.subcore_barrier()
    pltpu.sync_copy(out_sh.at[stripe], out_hbm.at[cid, stripe])


def _alpha_mat(a):
    a2 = a.reshape(H, C)
    eye = jnp.eye(H, dtype=_f32)
    return (a2[:, :, None] * eye[:, None, :]).reshape(H * C, H)


def kernel(x, pos, edge_index, batch, W1, b1, W2, b2,
           Wc0, al0, ar0, Wc1, al1, ar1, Wc2, al2, ar2,
           Wl1, bl1, Wl2, bl2):
    src = edge_index[0]
    dst = edge_index[1]
    pad = jnp.zeros((N, 5), _f32)
    hcat = jnp.concatenate([x, pos, pad], axis=1)
    w1t = jnp.concatenate([W1.T, jnp.zeros((5, HID), _f32)], axis=0)
    h = _encoder(hcat, w1t, b1.reshape(1, -1), W2.T, b2.reshape(1, -1))

    zeros_nh = jnp.zeros((NPAD, H), _f32)
    zpad_h = jnp.zeros((NPAD - N, H), _f32)
    p0 = p1 = None
    for li, (Wc, al, ar) in enumerate(
            ((Wc0, al0, ar0), (Wc1, al1, ar1), (Wc2, al2, ar2))):
        almat = _alpha_mat(al)
        armat = _alpha_mat(ar)
        if li == 0:
            xl, a_l, a_r = _pre_first(h, Wc.T, almat, armat)
        else:
            xl, a_l, a_r = _pre_next(p0, p1, Wc.T, almat, armat)
        a_l = jnp.concatenate([a_l, zpad_h], axis=0)
        a_r = jnp.concatenate([a_r, zpad_h], axis=0)
        ex, den2 = _sc_pass_a(a_l, a_r, src, dst, zeros_nh)
        p01 = _sc_pass_b(xl, ex, den2[0], den2[1], src, dst, zeros_nh)
        p0, p1 = p01[0, :N], p01[1, :N]

    batch3 = batch.reshape(NRB, 1, RB)
    return _decoder(p0, p1, batch3, Wl1.T, bl1.reshape(1, -1),
                    Wl2.T, bl2.reshape(1, -1))


# speedup vs baseline: 64.8072x; 1.3824x over previous
import functools

import jax
import jax.numpy as jnp
from jax import lax
from jax.experimental import pallas as pl
from jax.experimental.pallas import tpu as pltpu
from jax.experimental.pallas import tpu_sc as plsc

N = 10000
E = 160000
IN = 128
HID = 16
H = 16
C = 16
OUT = 128
NB = 64
NEG = 0.1

NC = 2
NS = 16
NW = NC * NS
K = 128
NCHUNK = E // K
NPAD = 10240
ZR = NPAD // NS

RB = 1000
NRB = N // RB

_f32 = jnp.float32


def _enc_body(xb, w1, b1, w2, b2, ob):
    t = jnp.maximum(
        jnp.dot(xb[...], w1[...], preferred_element_type=_f32) + b1[...], 0.0)
    ob[...] = jnp.dot(t, w2[...], preferred_element_type=_f32) + b2[...]


def _encoder(hcat, w1t, b1, w2t, b2):
    kin = hcat.shape[1]
    return pl.pallas_call(
        _enc_body,
        grid=(NRB,),
        in_specs=[
            pl.BlockSpec((RB, kin), lambda i: (i, 0)),
            pl.BlockSpec((kin, HID), lambda i: (0, 0)),
            pl.BlockSpec((1, HID), lambda i: (0, 0)),
            pl.BlockSpec((HID, HID), lambda i: (0, 0)),
            pl.BlockSpec((1, HID), lambda i: (0, 0)),
        ],
        out_specs=pl.BlockSpec((RB, HID), lambda i: (i, 0)),
        out_shape=jax.ShapeDtypeStruct((N, HID), _f32),
    )(hcat, w1t, b1, w2t, b2)


def _pre_first_body(hb, wc, al, ar, xl_o, al_o, ar_o):
    xl = jnp.dot(hb[...], wc[...], preferred_element_type=_f32)
    xl_o[...] = xl
    al_o[...] = jnp.dot(xl, al[...], preferred_element_type=_f32)
    ar_o[...] = jnp.dot(xl, ar[...], preferred_element_type=_f32)


def _pre_next_body(p0b, p1b, wc, al, ar, xl_o, al_o, ar_o):
    h = jnp.maximum(p0b[...] + p1b[...], 0.0)
    xl = jnp.dot(h, wc[...], preferred_element_type=_f32)
    xl_o[...] = xl
    al_o[...] = jnp.dot(xl, al[...], preferred_element_type=_f32)
    ar_o[...] = jnp.dot(xl, ar[...], preferred_element_type=_f32)


def _pre_specs(n_h_inputs):
    in_specs = [pl.BlockSpec((RB, HID), lambda i: (i, 0))] * n_h_inputs + [
        pl.BlockSpec((HID, H * C), lambda i: (0, 0)),
        pl.BlockSpec((H * C, H), lambda i: (0, 0)),
        pl.BlockSpec((H * C, H), lambda i: (0, 0)),
    ]
    out_specs = [
        pl.BlockSpec((RB, H * C), lambda i: (i, 0)),
        pl.BlockSpec((RB, H), lambda i: (i, 0)),
        pl.BlockSpec((RB, H), lambda i: (i, 0)),
    ]
    out_shape = [
        jax.ShapeDtypeStruct((N, H * C), _f32),
        jax.ShapeDtypeStruct((N, H), _f32),
        jax.ShapeDtypeStruct((N, H), _f32),
    ]
    return in_specs, out_specs, out_shape


def _pre_first(h, wct, almat, armat):
    ins, outs, oshape = _pre_specs(1)
    return pl.pallas_call(
        _pre_first_body, grid=(NRB,), in_specs=ins, out_specs=outs,
        out_shape=oshape)(h, wct, almat, armat)


def _pre_next(p0, p1, wct, almat, armat):
    ins, outs, oshape = _pre_specs(2)
    return pl.pallas_call(
        _pre_next_body, grid=(NRB,), in_specs=ins, out_specs=outs,
        out_shape=oshape)(p0, p1, wct, almat, armat)


def _dec_body(p0b, p1b, bb, w1, b1, w2, b2, ob):
    i = pl.program_id(0)
    h = jnp.maximum(p0b[...] + p1b[...], 0.0)
    t = jnp.maximum(
        jnp.dot(h, w1[...], preferred_element_type=_f32) + b1[...], 0.0)
    y = jnp.dot(t, w2[...], preferred_element_type=_f32) + b2[...]
    bidx = bb[0]
    oh = (lax.broadcasted_iota(jnp.int32, (NB, RB), 0) == bidx).astype(_f32)
    contrib = jnp.dot(oh, y, preferred_element_type=_f32)

    @pl.when(i == 0)
    def _():
        ob[...] = contrib

    @pl.when(i > 0)
    def _():
        ob[...] = ob[...] + contrib


def _decoder(p0, p1, batch3, w1t, b1, w2t, b2):
    return pl.pallas_call(
        _dec_body,
        grid=(NRB,),
        in_specs=[
            pl.BlockSpec((RB, HID), lambda i: (i, 0)),
            pl.BlockSpec((RB, HID), lambda i: (i, 0)),
            pl.BlockSpec((1, 1, RB), lambda i: (i, 0, 0)),
            pl.BlockSpec((HID, HID // 2), lambda i: (0, 0)),
            pl.BlockSpec((1, HID // 2), lambda i: (0, 0)),
            pl.BlockSpec((HID // 2, OUT), lambda i: (0, 0)),
            pl.BlockSpec((1, OUT), lambda i: (0, 0)),
        ],
        out_specs=pl.BlockSpec((NB, OUT), lambda i: (0, 0)),
        out_shape=jax.ShapeDtypeStruct((NB, OUT), _f32),
        compiler_params=pltpu.CompilerParams(
            dimension_semantics=("arbitrary",)),
    )(p0, p1, batch3, w1t, b1, w2t, b2)


_MESH = plsc.VectorSubcoreMesh(core_axis_name="c", subcore_axis_name="s")


@functools.partial(
    pl.kernel,
    out_type=(
        jax.ShapeDtypeStruct((E, H), _f32),
        jax.ShapeDtypeStruct((NC, NPAD, H), _f32),
    ),
    mesh=_MESH,
    compiler_params=pltpu.CompilerParams(use_tc_tiling_on_sc=False),
    scratch_types=[
        pltpu.VMEM((K,), jnp.int32),
        pltpu.VMEM((K,), jnp.int32),
        pltpu.VMEM((K, H), _f32),
        pltpu.VMEM((K, H), _f32),
        pltpu.VMEM_SHARED((NPAD, H), _f32),
        pltpu.VMEM_SHARED((NPAD, H), _f32),
        pltpu.VMEM_SHARED((NPAD, H), _f32),
    ],
)
def _sc_pass_a(al_hbm, ar_hbm, src_hbm, dst_hbm, zeros_hbm,
               ex_hbm, den_hbm,
               idx_s, idx_d, abuf, ebuf, al_sh, ar_sh, den_sh):
    cid = lax.axis_index("c")
    sid = lax.axis_index("s")
    wid = sid * NC + cid
    stripe = pl.ds(sid * ZR, ZR)

    pltpu.sync_copy(al_hbm.at[stripe], al_sh.at[stripe])
    pltpu.sync_copy(ar_hbm.at[stripe], ar_sh.at[stripe])
    pltpu.sync_copy(zeros_hbm.at[stripe], den_sh.at[stripe])
    plsc.subcore_barrier()

    nch = NCHUNK // NW + jnp.where(wid < (NCHUNK % NW), 1, 0)

    def chunk(j, carry):
        base = (wid + NW * j) * K
        pltpu.sync_copy(src_hbm.at[pl.ds(base, K)], idx_s)
        pltpu.sync_copy(dst_hbm.at[pl.ds(base, K)], idx_d)
        pltpu.sync_copy(al_sh.at[idx_s], abuf)
        pltpu.sync_copy(ar_sh.at[idx_d], ebuf)

        @plsc.parallel_loop(0, K, unroll=8)
        def edge(e):
            v = abuf[e] + ebuf[e]
            v = jnp.maximum(v, v * NEG)
            ebuf[e] = jnp.exp(v)
        pltpu.sync_copy(ebuf, ex_hbm.at[pl.ds(base, K)])
        pltpu.sync_copy(ebuf, den_sh.at[idx_d], add=True)
        return carry

    lax.fori_loop(0, nch, chunk, 0)
    plsc.subcore_barrier()
    pltpu.sync_copy(den_sh.at[stripe], den_hbm.at[cid, stripe])


@functools.partial(
    pl.kernel,
    out_type=jax.ShapeDtypeStruct((NC, NPAD, C), _f32),
    mesh=_MESH,
    compiler_params=pltpu.CompilerParams(use_tc_tiling_on_sc=False),
    scratch_types=[
        pltpu.VMEM((K,), jnp.int32),
        pltpu.VMEM((K,), jnp.int32),
        pltpu.VMEM((K, H), _f32),
        pltpu.VMEM((K, H), _f32),
        pltpu.VMEM((K, H * C), _f32),
        pltpu.VMEM((K, C), _f32),
        pltpu.VMEM((ZR, H), _f32),
        pltpu.VMEM((ZR, H), _f32),
        pltpu.SemaphoreType.DMA,
        pltpu.VMEM_SHARED((NPAD, H), _f32),
        pltpu.VMEM_SHARED((NPAD, C), _f32),
    ],
)
def _sc_pass_b(xl_hbm, ex_hbm, den0_hbm, den1_hbm, src_hbm, dst_hbm,
               zeros_hbm, out_hbm,
               idx_s, idx_d, exbuf, dbuf, xlb, mb, v0, v1,
               sem1, den_sh, out_sh):
    cid = lax.axis_index("c")
    sid = lax.axis_index("s")
    wid = sid * NC + cid
    stripe = pl.ds(sid * ZR, ZR)

    pltpu.sync_copy(den0_hbm.at[stripe], v0)
    pltpu.sync_copy(den1_hbm.at[stripe], v1)

    @plsc.parallel_loop(0, ZR, unroll=8)
    def addrow(e):
        v0[e] = 1.0 / (v0[e] + v1[e] + 1e-16)

    pltpu.sync_copy(v0, den_sh.at[stripe])
    pltpu.sync_copy(zeros_hbm.at[stripe], out_sh.at[stripe])
    plsc.subcore_barrier()

    nch = NCHUNK // NW + jnp.where(wid < (NCHUNK % NW), 1, 0)

    def chunk(j, carry):
        base = (wid + NW * j) * K
        pltpu.sync_copy(src_hbm.at[pl.ds(base, K)], idx_s)
        pltpu.sync_copy(dst_hbm.at[pl.ds(base, K)], idx_d)
        g1 = pltpu.async_copy(xl_hbm.at[idx_s], xlb, sem1)
        pltpu.sync_copy(ex_hbm.at[pl.ds(base, K)], exbuf)
        pltpu.sync_copy(den_sh.at[idx_d], dbuf)
        g1.wait()

        @plsc.parallel_loop(0, K, unroll=2)
        def edge(e):
            w = exbuf[e] * dbuf[e]
            acc = w[0] * xlb[e, pl.ds(0, C)]
            for h in range(1, H):
                acc = acc + w[h] * xlb[e, pl.ds(h * C, C)]
            mb[e] = acc * (1.0 / H)
        pltpu.sync_copy(mb, out_sh.at[idx_d], add=True)
        return carry

    lax.fori_loop(0, nch, chunk, 0)
    plsc.subcore_barrier()
    pltpu.sync_copy(out_sh.at[stripe], out_hbm.at[cid, stripe])


def _alpha_mat(a):
    a2 = a.reshape(H, C)
    eye = jnp.eye(H, dtype=_f32)
    return (a2[:, :, None] * eye[:, None, :]).reshape(H * C, H)


def kernel(x, pos, edge_index, batch, W1, b1, W2, b2,
           Wc0, al0, ar0, Wc1, al1, ar1, Wc2, al2, ar2,
           Wl1, bl1, Wl2, bl2):
    src = edge_index[0]
    dst = edge_index[1]
    pad = jnp.zeros((N, 5), _f32)
    hcat = jnp.concatenate([x, pos, pad], axis=1)
    w1t = jnp.concatenate([W1.T, jnp.zeros((5, HID), _f32)], axis=0)
    h = _encoder(hcat, w1t, b1.reshape(1, -1), W2.T, b2.reshape(1, -1))

    zeros_nh = jnp.zeros((NPAD, H), _f32)
    zpad_h = jnp.zeros((NPAD - N, H), _f32)
    p0 = p1 = None
    for li, (Wc, al, ar) in enumerate(
            ((Wc0, al0, ar0), (Wc1, al1, ar1), (Wc2, al2, ar2))):
        almat = _alpha_mat(al)
        armat = _alpha_mat(ar)
        if li == 0:
            xl, a_l, a_r = _pre_first(h, Wc.T, almat, armat)
        else:
            xl, a_l, a_r = _pre_next(p0, p1, Wc.T, almat, armat)
        a_l = jnp.concatenate([a_l, zpad_h], axis=0)
        a_r = jnp.concatenate([a_r, zpad_h], axis=0)
        ex, den2 = _sc_pass_a(a_l, a_r, src, dst, zeros_nh)
        p01 = _sc_pass_b(xl, ex, den2[0], den2[1], src, dst, zeros_nh)
        p0, p1 = p01[0, :N], p01[1, :N]

    batch3 = batch.reshape(NRB, 1, RB)
    return _decoder(p0, p1, batch3, Wl1.T, bl1.reshape(1, -1),
                    Wl2.T, bl2.reshape(1, -1))
